# baseline (speedup 1.0000x reference)
import jax
import jax.numpy as jnp
from jax.experimental import pallas as pl
from jax.experimental.pallas import tpu as pltpu

F_DIM = 128
K_NEG = 65536
B = 1024

_COPY_BLK = 16384
_GRID = K_NEG // _COPY_BLK
_LROWS = K_NEG // 128 // _GRID
_BROWS = B // 128


def _merge_body(p_ref, buf_blk, keys_blk, mlab_blk, lab_blk,
                out_blk, lout_blk, pout_blk):
    i = pl.program_id(0)
    out_blk[...] = buf_blk[...]
    lout_blk[...] = mlab_blk[...]
    p_raw = p_ref[0]
    p = jnp.clip(jnp.where(p_raw < 0, p_raw + K_NEG, p_raw), 0, K_NEG - B)

    @pl.when(i == p // _COPY_BLK)
    def _():
        off = pl.multiple_of(p - (p // _COPY_BLK) * _COPY_BLK, B)
        out_blk[:, pl.ds(off, B)] = jnp.transpose(keys_blk[...], (1, 0))
        roff = pl.multiple_of(off // 128, _BROWS)
        lout_blk[pl.ds(roff, _BROWS), :] = lab_blk[...]

    @pl.when(i == 0)
    def _():
        pout_blk[...] = jnp.full((1, 1), (p_ref[0] + B) % K_NEG, jnp.int32)


_tc_merge = pl.pallas_call(
    _merge_body,
    out_shape=(
        jax.ShapeDtypeStruct((F_DIM, K_NEG), jnp.float32),
        jax.ShapeDtypeStruct((K_NEG // 128, 128), jnp.int32),
        jax.ShapeDtypeStruct((1, 1), jnp.int32),
    ),
    grid_spec=pltpu.PrefetchScalarGridSpec(
        num_scalar_prefetch=1,
        grid=(_GRID,),
        in_specs=[
            pl.BlockSpec((F_DIM, _COPY_BLK), lambda i, p: (0, i)),
            pl.BlockSpec((B, F_DIM), lambda i, p: (0, 0)),
            pl.BlockSpec((_LROWS, 128), lambda i, p: (i, 0)),
            pl.BlockSpec((_BROWS, 128), lambda i, p: (0, 0)),
        ],
        out_specs=(
            pl.BlockSpec((F_DIM, _COPY_BLK), lambda i, p: (0, i)),
            pl.BlockSpec((_LROWS, 128), lambda i, p: (i, 0)),
            pl.BlockSpec((1, 1), lambda i, p: (0, 0)),
        ),
    ),
)


import functools
from jax import lax
from jax.experimental.pallas import tpu_sc as plsc

LANES = 16
_mesh = plsc.VectorSubcoreMesh(core_axis_name="c", subcore_axis_name="s")


@functools.partial(
    pl.kernel,
    out_type=jax.ShapeDtypeStruct((1,), jnp.int32),
    mesh=_mesh,
    compiler_params=pltpu.CompilerParams(needs_layout_passes=False),
    scratch_types=[
        pltpu.VMEM((1,), jnp.int32),
        pltpu.VMEM((LANES,), jnp.int32),
    ],
)
def _sc_ptr(ptr_hbm, ptr_out, ptr_v, nptr_v):
    wid = lax.axis_index("s") * 2 + lax.axis_index("c")

    @pl.when(wid == 0)
    def _():
        pltpu.sync_copy(ptr_hbm, ptr_v)
        pvec = plsc.load_gather(ptr_v, [jnp.zeros((LANES,), jnp.int32)])
        nptr_v[...] = lax.rem(pvec + B, K_NEG)
        pltpu.sync_copy(nptr_v.at[pl.ds(0, 1)], ptr_out)


def kernel(keys, labels, buffer, mem_labels, ptr):
    new_buffer, lab2d, _ = _tc_merge(
        ptr, buffer, keys,
        mem_labels.reshape(K_NEG // 128, 128),
        labels.reshape(_BROWS, 128))
    new_ptr = _sc_ptr(ptr)
    return new_buffer, lab2d.reshape(K_NEG), new_ptr


# speedup vs baseline: 1.6411x; 1.6411x over previous
import jax
import jax.numpy as jnp
from jax.experimental import pallas as pl
from jax.experimental.pallas import tpu as pltpu

F_DIM = 128
K_NEG = 65536
B = 1024

_COPY_BLK = 16384
_GRID = K_NEG // _COPY_BLK
_LROWS = K_NEG // 128 // _GRID
_BROWS = B // 128


def _merge_body(p_ref, buf_blk, keys_blk, mlab_blk, lab_blk,
                out_blk, lout_blk, pout_blk):
    i = pl.program_id(0)
    out_blk[...] = buf_blk[...]
    lout_blk[...] = mlab_blk[...]
    p_raw = p_ref[0]
    p = jnp.clip(jnp.where(p_raw < 0, p_raw + K_NEG, p_raw), 0, K_NEG - B)

    @pl.when(i == p // _COPY_BLK)
    def _():
        off = pl.multiple_of(p - (p // _COPY_BLK) * _COPY_BLK, B)
        out_blk[:, pl.ds(off, B)] = jnp.transpose(keys_blk[...], (1, 0))
        roff = pl.multiple_of(off // 128, _BROWS)
        lout_blk[pl.ds(roff, _BROWS), :] = lab_blk[...]

    @pl.when(i == 0)
    def _():
        pout_blk[...] = jnp.full((1, 1), (p_ref[0] + B) % K_NEG, jnp.int32)


_tc_merge = pl.pallas_call(
    _merge_body,
    out_shape=(
        jax.ShapeDtypeStruct((F_DIM, K_NEG), jnp.float32),
        jax.ShapeDtypeStruct((K_NEG // 128, 128), jnp.int32),
        jax.ShapeDtypeStruct((1, 1), jnp.int32),
    ),
    grid_spec=pltpu.PrefetchScalarGridSpec(
        num_scalar_prefetch=1,
        grid=(_GRID,),
        in_specs=[
            pl.BlockSpec((F_DIM, _COPY_BLK), lambda i, p: (0, i)),
            pl.BlockSpec((B, F_DIM), lambda i, p: (0, 0)),
            pl.BlockSpec((_LROWS, 128), lambda i, p: (i, 0)),
            pl.BlockSpec((_BROWS, 128), lambda i, p: (0, 0)),
        ],
        out_specs=(
            pl.BlockSpec((F_DIM, _COPY_BLK), lambda i, p: (0, i)),
            pl.BlockSpec((_LROWS, 128), lambda i, p: (i, 0)),
            pl.BlockSpec((1, 1), lambda i, p: (0, 0)),
        ),
    ),
)


def kernel(keys, labels, buffer, mem_labels, ptr):
    new_buffer, lab2d, nptr = _tc_merge(
        ptr, buffer, keys,
        mem_labels.reshape(K_NEG // 128, 128),
        labels.reshape(_BROWS, 128))
    return new_buffer, lab2d.reshape(K_NEG), nptr.reshape(1)
